# baseline (speedup 1.0000x reference)
import functools

import jax
import jax.numpy as jnp
from jax import lax
from jax.experimental import pallas as pl
from jax.experimental.pallas import tpu as pltpu
from jax.experimental.pallas import tpu_sc as plsc


_V = 21
_LANES = 16


def _pad16(n):
    return (n + _LANES - 1) // _LANES * _LANES


def _sc_histogram(x):
    B, L = x.shape
    info = plsc.get_sparse_core_info()
    NC, NS = info.num_cores, info.num_subcores
    NW = NC * NS
    rows = B // NW
    nfull = L // _LANES
    CP = _pad16(_V * L)
    mesh = plsc.VectorSubcoreMesh(core_axis_name="c", subcore_axis_name="s")

    @functools.partial(
        pl.kernel,
        out_type=jax.ShapeDtypeStruct((NW, CP), jnp.float32),
        mesh=mesh,
        scratch_types=[
            pltpu.VMEM((rows * L,), jnp.int32),
            pltpu.VMEM((CP,), jnp.float32),
        ],
        compiler_params=pltpu.CompilerParams(needs_layout_passes=False),
    )
    def hist(x_hbm, out_hbm, xbuf, cnt):
        wid = lax.axis_index("s") * NC + lax.axis_index("c")
        pltpu.sync_copy(x_hbm.at[pl.ds(wid * rows * L, rows * L)], xbuf)

        zeros = jnp.zeros((_LANES,), jnp.float32)
        for j in range(CP // _LANES):
            cnt[pl.ds(j * _LANES, _LANES)] = zeros

        ones = jnp.ones((_LANES,), jnp.float32)
        lane = lax.iota(jnp.int32, _LANES)
        tail_mask = lane >= (_LANES - L % _LANES)

        def row_body(r, carry):
            base = r * L
            for j in range(nfull):
                xv = xbuf[pl.ds(base + j * _LANES, _LANES)]
                plsc.addupdate_scatter(
                    cnt, [xv * L + (j * _LANES + lane)], ones)
            xv = xbuf[pl.ds(base + (L - _LANES), _LANES)]
            plsc.addupdate_scatter(
                cnt, [xv * L + ((L - _LANES) + lane)], ones, mask=tail_mask)
            return carry

        lax.fori_loop(0, rows, row_body, 0)
        pltpu.sync_copy(cnt, out_hbm.at[wid])

    return hist(x.reshape(-1))


def _dense_body(p_ref, t_ref, w1_ref, b1_ref, w2_ref, b2_ref, out_ref, a_ref):
    cT = jnp.sum(p_ref[...], axis=0)
    v_ids = lax.broadcasted_iota(jnp.int32, cT.shape, 0)
    cT = jnp.where(v_ids == 0, 0.0, cT)
    a = lax.dot_general(cT, t_ref[...], (((0,), (0,)), ((), ())),
                        preferred_element_type=jnp.float32)
    a_ref[...] = a
    h = jnp.tanh(
        lax.dot(a, w1_ref[...], preferred_element_type=jnp.float32)
        + b1_ref[...])
    out_ref[...] = (
        lax.dot(h, w2_ref[...], preferred_element_type=jnp.float32)
        + b2_ref[...])


def kernel(x, table, W1, b1, W2, b2):
    B, L = x.shape
    V, D = table.shape
    H = W1.shape[1]

    partials = _sc_histogram(x)
    partials = partials[:, :V * L].reshape(-1, V, L)

    out, a = pl.pallas_call(
        _dense_body,
        out_shape=[
            jax.ShapeDtypeStruct((L, D), jnp.float32),
            jax.ShapeDtypeStruct((L, D), jnp.float32),
        ],
    )(partials, table, W1, b1.reshape(1, H), W2, b2.reshape(1, D))
    return (out, a)


# speedup vs baseline: 1.0424x; 1.0424x over previous
import functools

import jax
import jax.numpy as jnp
from jax import lax
from jax.experimental import pallas as pl
from jax.experimental.pallas import tpu as pltpu
from jax.experimental.pallas import tpu_sc as plsc


_V = 21
_LANES = 16


def _pad16(n):
    return (n + _LANES - 1) // _LANES * _LANES


def _sc_histogram(x):
    B, L = x.shape
    info = plsc.get_sparse_core_info()
    NC, NS = info.num_cores, info.num_subcores
    NW = NC * NS
    rows = B // NW
    nfull = L // _LANES
    LP = 256
    CP = _V * LP
    mesh = plsc.VectorSubcoreMesh(core_axis_name="c", subcore_axis_name="s")

    @functools.partial(
        pl.kernel,
        out_type=jax.ShapeDtypeStruct((NW, CP), jnp.float32),
        mesh=mesh,
        scratch_types=[
            pltpu.VMEM((rows * L,), jnp.int32),
            pltpu.VMEM((CP,), jnp.float32),
            pltpu.SemaphoreType.DMA,
        ],
        compiler_params=pltpu.CompilerParams(needs_layout_passes=False),
    )
    def hist(x_hbm, out_hbm, xbuf, cnt, sem):
        wid = lax.axis_index("s") * NC + lax.axis_index("c")
        cp_in = pltpu.async_copy(
            x_hbm.at[pl.ds(wid * rows * L, rows * L)], xbuf, sem)

        zeros = jnp.zeros((_LANES,), jnp.float32)
        for j in range(CP // _LANES):
            cnt[pl.ds(j * _LANES, _LANES)] = zeros

        ones = jnp.ones((_LANES,), jnp.float32)
        lane = lax.iota(jnp.int32, _LANES)
        tail_mask = lane >= (_LANES - L % _LANES)
        cp_in.wait()

        def row_group_body(g, carry):
            for u in range(4):
                base = (g * 4 + u) * L
                for j in range(nfull):
                    xv = xbuf[pl.ds(base + j * _LANES, _LANES)]
                    plsc.addupdate_scatter(
                        cnt, [xv * LP + (j * _LANES + lane)], ones)
                xv = xbuf[pl.ds(base + (L - _LANES), _LANES)]
                plsc.addupdate_scatter(
                    cnt, [xv * LP + ((L - _LANES) + lane)], ones,
                    mask=tail_mask)
            return carry

        lax.fori_loop(0, rows // 4, row_group_body, 0)

        pltpu.sync_copy(cnt, out_hbm.at[wid])

    return hist(x.reshape(-1))


def _dense_body(p_ref, t_ref, w1_ref, b1_ref, w2_ref, b2_ref, out_ref, a_ref):
    L = out_ref.shape[0]
    cT = jnp.sum(p_ref[...], axis=0)[:, :L]
    v_ids = lax.broadcasted_iota(jnp.int32, cT.shape, 0)
    cT = jnp.where(v_ids == 0, 0.0, cT)
    a = lax.dot_general(cT, t_ref[...], (((0,), (0,)), ((), ())),
                        preferred_element_type=jnp.float32)
    a_ref[...] = a
    h = jnp.tanh(
        lax.dot(a, w1_ref[...], preferred_element_type=jnp.float32)
        + b1_ref[...])
    out_ref[...] = (
        lax.dot(h, w2_ref[...], preferred_element_type=jnp.float32)
        + b2_ref[...])


def kernel(x, table, W1, b1, W2, b2):
    B, L = x.shape
    V, D = table.shape
    H = W1.shape[1]

    partials = _sc_histogram(x).reshape(-1, V, 256)

    out, a = pl.pallas_call(
        _dense_body,
        out_shape=[
            jax.ShapeDtypeStruct((L, D), jnp.float32),
            jax.ShapeDtypeStruct((L, D), jnp.float32),
        ],
    )(partials, table, W1, b1.reshape(1, H), W2, b2.reshape(1, D))
    return (out, a)
